# Initial kernel scaffold; baseline (speedup 1.0000x reference)
import jax
import jax.numpy as jnp
from jax.experimental import pallas as pl


def kernel(edge_index, senders_pos, receivers_pos, edge_dx_, edge_attr, vector_a, vector_b, vector_c, senders_v_t_, senders_v_tm1_, senders_w_t_, receivers_v_t_, receivers_v_tm1_, receivers_w_t_, node_a_t1, node_alpha_t1, node_latent, params):
    raise NotImplementedError("write your pallas kernel here")


# speedup vs baseline: 2.0777x; 2.0777x over previous
import functools

import jax
import jax.numpy as jnp
from jax import lax
from jax.experimental import pallas as pl
from jax.experimental.pallas import tpu as pltpu
from jax.experimental.pallas import tpu_sc as plsc

F32 = jnp.float32
LAT = 128
NN = 10000
NE = 320000
EB = 1280
NB = 2000
NWORK = 32
EPW = NE // NWORK
CH = 80
NCHUNK = EPW // CH


def _relu(x):
    return jnp.maximum(x, 0.0)


def _ln(x, g, b):
    mu = jnp.mean(x, axis=-1, keepdims=True)
    var = jnp.mean((x - mu) ** 2, axis=-1, keepdims=True)
    return (x - mu) / jnp.sqrt(var + 1e-5) * g + b


def _mm(x, w):
    return lax.dot_general(x, w, (((1,), (0,)), ((), ())),
                           preferred_element_type=F32)


def _mmT(xt, w):
    return lax.dot_general(xt, w, (((0,), (0,)), ((), ())),
                           preferred_element_type=F32)


def _mmO(w, h):
    return lax.dot_general(w, h, (((0,), (1,)), ((), ())),
                           preferred_element_type=F32)


def _node_body(nl_ref, na_ref, nal_ref,
               nw0, nb0, nw1, nb1, nw2, nb2,
               mw0, mb0, mw1, mb1, mw2, mb2,
               iw0, ib0, iw1, ib1, iw2, ib2,
               fw0, fb0, fw1, fb1, fw2, fb2,
               tw0, tb0, tw1, tb1, tw2, tb2,
               aux_ref):
    x = nl_ref[...]

    def dec(w0, b0, w1, b1, w2, b2):
        h = _relu(_mm(x, w0[...]) + b0[...])
        h = _relu(_mm(h, w1[...]) + b1[...])
        return _mm(h, w2[...]) + b2[...]

    w = dec(nw0, nb0, nw1, nb1, nw2, nb2)
    m = dec(mw0, mb0, mw1, mb1, mw2, mb2)
    mi = dec(iw0, ib0, iw1, ib1, iw2, ib2)
    fx = dec(fw0, fb0, fw1, fb1, fw2, fb2)
    tx = dec(tw0, tb0, tw1, tb1, tw2, tb2)
    bl = m * na_ref[...] - fx
    ba = mi * nal_ref[...] - tx
    aux_ref[...] = jnp.concatenate([w, bl, ba, jnp.zeros_like(w)], axis=1)


def _node_stage(node_latent, node_a_t1, node_alpha_t1, decs):
    grid = NN // NB
    full = lambda a: pl.BlockSpec(a.shape, lambda i: (0,) * a.ndim)
    win = [
        pl.BlockSpec((NB, LAT), lambda i: (i, 0)),
        pl.BlockSpec((NB, 3), lambda i: (i, 0)),
        pl.BlockSpec((NB, 3), lambda i: (i, 0)),
    ]
    flat = []
    for d in decs:
        flat += d
    win += [full(a) for a in flat]
    return pl.pallas_call(
        _node_body,
        grid=(grid,),
        in_specs=win,
        out_specs=pl.BlockSpec((NB, 8), lambda i: (i, 0)),
        out_shape=jax.ShapeDtypeStruct((NN, 8), F32),
    )(node_latent, node_a_t1, node_alpha_t1, *flat)


def _sc_gather(g, w, snd, rcv):
    mesh = plsc.VectorSubcoreMesh(core_axis_name="c", subcore_axis_name="s")

    @functools.partial(
        pl.kernel, mesh=mesh,
        out_type=[jax.ShapeDtypeStruct((NE, LAT), F32),
                  jax.ShapeDtypeStruct((NE,), F32),
                  jax.ShapeDtypeStruct((NE,), F32)],
        scratch_types=[pltpu.VMEM((CH,), jnp.int32),
                       pltpu.VMEM((CH,), jnp.int32),
                       pltpu.VMEM((CH, LAT), F32),
                       pltpu.VMEM((CH, LAT), F32),
                       pltpu.VMEM((NN,), F32),
                       pltpu.VMEM((CH,), F32),
                       pltpu.VMEM((CH,), F32),
                       pltpu.SemaphoreType.DMA],
        compiler_params=pltpu.CompilerParams(needs_layout_passes=False),
    )
    def k(g_hbm, w_hbm, snd_hbm, rcv_hbm, ge_hbm, ws_hbm, wr_hbm,
          idx_s, idx_r, rows_s, rows_r, wbuf, wsv, wrv, sem):
        wid = lax.axis_index("s") * 2 + lax.axis_index("c")
        base0 = wid * EPW
        pltpu.sync_copy(w_hbm, wbuf)

        def chunk(kk, carry):
            base = base0 + kk * CH
            pltpu.sync_copy(snd_hbm.at[pl.ds(base, CH)], idx_s)
            pltpu.sync_copy(rcv_hbm.at[pl.ds(base, CH)], idx_r)
            cp1 = pltpu.async_copy(g_hbm.at[idx_s], rows_s, sem)
            cp2 = pltpu.async_copy(g_hbm.at[idx_r], rows_r, sem)
            cp1.wait()
            cp2.wait()

            def addrow(i, c2):
                for j in range(LAT // 16):
                    sl = pl.ds(j * 16, 16)
                    rows_s[i, sl] = rows_s[i, sl] + rows_r[i, sl]
                return c2

            lax.fori_loop(0, CH, addrow, 0)
            for k2 in range(CH // 16):
                sl = pl.ds(k2 * 16, 16)
                wsv[sl] = plsc.load_gather(wbuf, [idx_s[sl]])
                wrv[sl] = plsc.load_gather(wbuf, [idx_r[sl]])
            pltpu.sync_copy(rows_s, ge_hbm.at[pl.ds(base, CH)])
            pltpu.sync_copy(wsv, ws_hbm.at[pl.ds(base, CH)])
            pltpu.sync_copy(wrv, wr_hbm.at[pl.ds(base, CH)])
            return carry

        lax.fori_loop(0, NCHUNK, chunk, 0)

    return k(g, w, snd, rcv)


NNP = 10240
TBL = NNP * 8
NQ = 8
QL = TBL // NQ
SLC = QL // 16
VPW = NE * 8 // NWORK
SCH = 2000
NSCH = VPW // SCH


def _sc_scatter(vals, offs):
    mesh = plsc.VectorSubcoreMesh(core_axis_name="c", subcore_axis_name="s")

    @functools.partial(
        pl.kernel, mesh=mesh,
        out_type=jax.ShapeDtypeStruct((2, TBL), F32),
        scratch_types=[pltpu.VMEM((SCH,), F32),
                       pltpu.VMEM((SCH,), jnp.int32),
                       pltpu.VMEM((TBL,), F32),
                       pltpu.VMEM((SLC,), F32),
                       pltpu.VMEM((SLC,), F32),
                       pltpu.VMEM_SHARED((16, QL), F32),
                       pltpu.SemaphoreType.DMA],
        compiler_params=pltpu.CompilerParams(needs_layout_passes=False),
    )
    def k(val_hbm, off_hbm, part_hbm, vbuf, obuf, tbl, acc, tmp, shared, sem):
        c = lax.axis_index("c")
        s = lax.axis_index("s")
        wid = s * 2 + c
        base0 = wid * VPW
        z16 = jnp.zeros((16,), F32)

        def zero(i, carry):
            tbl[pl.ds(i * 16, 16)] = z16
            return carry

        lax.fori_loop(0, TBL // 16, zero, 0)

        def chunk(kk, carry):
            base = base0 + kk * SCH
            pltpu.sync_copy(val_hbm.at[pl.ds(base, SCH)], vbuf)
            pltpu.sync_copy(off_hbm.at[pl.ds(base, SCH)], obuf)

            def scat16(i, c2):
                sl = pl.ds(i * 16, 16)
                plsc.addupdate_scatter(tbl, [obuf[sl]], vbuf[sl])
                return c2

            lax.fori_loop(0, SCH // 16, scat16, 0)
            return carry

        lax.fori_loop(0, NSCH, chunk, 0)

        def red(i, carry):
            sl = pl.ds(i * 16, 16)
            acc[sl] = acc[sl] + tmp[sl]
            return carry

        for q in range(NQ):
            qoff = q * QL
            pltpu.sync_copy(tbl.at[pl.ds(qoff, QL)], shared.at[s])
            plsc.subcore_barrier()
            myslc = pl.ds(s * SLC, SLC)
            pltpu.sync_copy(shared.at[0, myslc], acc)
            for t in range(1, 16):
                pltpu.sync_copy(shared.at[t, myslc], tmp)
                lax.fori_loop(0, SLC // 16, red, 0)
            pltpu.sync_copy(acc, part_hbm.at[c, pl.ds(qoff + s * SLC, SLC)])
            plsc.subcore_barrier()

    return k(vals, offs)


def _edge_body(sv, ge,
               ew0, eb0, ew1, eb1, ew2, eb2, elg, elb,
               nw0, nb0, nw1, nb1, nw2, nb2, nlg, nlb,
               wa, wb, wc, ib0, iw1, ib1, iw2, ib2, ilg, ilb,
               aw0, ab0, aw1, ab1, aw2, ab2,
               cw0, cb0, cw1, cb1, cw2, cb2,
               sw0, sb0, sw1, sb1, sw2, sb2,
               il_ref, o8_ref):
    r = lambda i: sv[i:i + 1, :]

    def dot3(a0, a1, a2, b0, b1, b2):
        return a0 * b0 + a1 * b1 + a2 * b2

    def project(v0, v1, v2, sign):
        p0 = dot3(r(0), r(1), r(2), v0, v1, v2) * sign
        p1 = dot3(r(3), r(4), r(5), v0, v1, v2) * sign
        p2 = dot3(r(6), r(7), r(8), v0, v1, v2) * sign
        return p0, p1, p2

    sv0, sv1, sv2 = project(r(9), r(10), r(11), 1.0)
    sw_0, sw_1, sw_2 = project(r(12), r(13), r(14), 1.0)
    rv0, rv1, rv2 = project(r(15), r(16), r(17), -1.0)
    rw_0, rw_1, rw_2 = project(r(18), r(19), r(20), -1.0)

    z = jnp.zeros_like(sv0)
    sfT = jnp.concatenate([sv0, sv1, sv2, sw_0, sw_1, sw_2, z, z], axis=0)
    rfT = jnp.concatenate([rv0, rv1, rv2, rw_0, rw_1, rw_2, z, z], axis=0)
    nrm = jnp.sqrt(r(21) * r(21) + r(22) * r(22) + r(23) * r(23))
    efT = jnp.concatenate([nrm, sv[24:28, :], z, z, z], axis=0)

    def enc(xT, w0, b0, w1, b1, w2, b2, lg, lb):
        h = _relu(_mmT(xT, w0[...]) + b0[...])
        h = _relu(_mm(h, w1[...]) + b1[...])
        h = _mm(h, w2[...]) + b2[...]
        return _ln(h, lg[...], lb[...])

    sp = enc(sfT, nw0, nb0, nw1, nb1, nw2, nb2, nlg, nlb)
    rp = enc(rfT, nw0, nb0, nw1, nb1, nw2, nb2, nlg, nlb)
    el = enc(efT, ew0, eb0, ew1, eb1, ew2, eb2, elg, elb)

    x1 = _relu(_mm(sp + rp, wa[...]) + _mm(ge[...], wb[...])
               + _mm(el, wc[...]) + ib0[...])
    x2 = _relu(_mm(x1, iw1[...]) + ib1[...])
    il = _ln(_mm(x2, iw2[...]) + ib2[...], ilg[...], ilb[...])
    il_ref[...] = il

    def dec(w0, b0, w1, b1, w2, b2):
        h = _relu(_mm(il, w0[...]) + b0[...])
        h = _relu(_mm(h, w1[...]) + b1[...])
        return _mmO(w2[...], h) + b2[...]

    cf = dec(aw0, ab0, aw1, ab1, aw2, ab2)
    ca = dec(cw0, cb0, cw1, cb1, cw2, cb2)
    lam = dec(sw0, sb0, sw1, sb1, sw2, sb2)[0:1, :]

    def combo(cT, i):
        c0, c1, c2 = cT[0:1, :], cT[1:2, :], cT[2:3, :]
        return c0 * r(i) + c1 * r(3 + i) + c2 * r(6 + i)

    f0, f1, f2 = combo(cf, 0), combo(cf, 1), combo(cf, 2)
    a0, a1, a2 = combo(ca, 0), combo(ca, 1), combo(ca, 2)

    ws_ = r(34)
    wr_ = r(35)
    den = ws_ + wr_
    lv = []
    for i in range(3):
        r0 = (ws_ * r(28 + i) + wr_ * r(31 + i)) / den
        lv.append(r(31 + i) - r0)
    g0, g1, g2 = f0 * lam, f1 * lam, f2 * lam
    t0 = lv[1] * g2 - lv[2] * g1
    t1 = lv[2] * g0 - lv[0] * g2
    t2 = lv[0] * g1 - lv[1] * g0
    o8_ref[...] = jnp.concatenate(
        [f0, f1, f2, a0 - t0, a1 - t1, a2 - t2, z, z], axis=0)


def _edge_stage(sv, ge, wlist):
    grid = NE // EB
    full = lambda a: pl.BlockSpec(a.shape, lambda i: (0,) * a.ndim)
    in_specs = [pl.BlockSpec((40, EB), lambda i: (0, i)),
                pl.BlockSpec((EB, LAT), lambda i: (i, 0))]
    in_specs += [full(a) for a in wlist]
    return pl.pallas_call(
        _edge_body,
        grid=(grid,),
        in_specs=in_specs,
        out_specs=[pl.BlockSpec((EB, LAT), lambda i: (i, 0)),
                   pl.BlockSpec((8, EB), lambda i: (0, i))],
        out_shape=[jax.ShapeDtypeStruct((NE, LAT), F32),
                   jax.ShapeDtypeStruct((8, NE), F32)],
    )(sv, ge, *wlist)


def _fin_body(part_ref, aux_ref, rl_ref, ra_ref):
    p0 = part_ref[0]
    p1 = part_ref[1]
    rl_ref[...] = aux_ref[:, 1:4] - p0[:, 0:3] - p1[:, 0:3]
    ra_ref[...] = aux_ref[:, 4:7] - p0[:, 3:6] - p1[:, 3:6]


def _finalize(part, aux):
    grid = NN // NB
    return pl.pallas_call(
        _fin_body,
        grid=(grid,),
        in_specs=[pl.BlockSpec((2, NB, 8), lambda i: (0, i, 0)),
                  pl.BlockSpec((NB, 8), lambda i: (i, 0))],
        out_specs=[pl.BlockSpec((NB, 3), lambda i: (i, 0)),
                   pl.BlockSpec((NB, 3), lambda i: (i, 0))],
        out_shape=[jax.ShapeDtypeStruct((NN, 3), F32),
                   jax.ShapeDtypeStruct((NN, 3), F32)],
    )(part, aux)


def _row_bias(b):
    return b.reshape(1, -1)


def _enc_weights(p):
    w0 = p["W"][0]
    w0p = jnp.zeros((8, LAT), F32).at[: w0.shape[0]].set(w0)
    return [w0p, _row_bias(p["b"][0]), p["W"][1], _row_bias(p["b"][1]),
            p["W"][2], _row_bias(p["b"][2]),
            _row_bias(p["ln_g"]), _row_bias(p["ln_b"])]


def _dec_weights(p):
    w2 = p["W"][2]
    w2p = jnp.zeros((LAT, 8), F32).at[:, : w2.shape[1]].set(w2)
    b2p = jnp.zeros((8,), F32).at[: w2.shape[1]].set(p["b"][2]).reshape(8, 1)
    return [p["W"][0], _row_bias(p["b"][0]),
            p["W"][1], _row_bias(p["b"][1]), w2p, b2p]


def _dec_weights_n(p):
    return [p["W"][0], _row_bias(p["b"][0]), p["W"][1], _row_bias(p["b"][1]),
            p["W"][2], _row_bias(p["b"][2])]


def kernel(edge_index, senders_pos, receivers_pos, edge_dx_, edge_attr,
           vector_a, vector_b, vector_c, senders_v_t_, senders_v_tm1_,
           senders_w_t_, receivers_v_t_, receivers_v_tm1_, receivers_w_t_,
           node_a_t1, node_alpha_t1, node_latent, params):
    senders = edge_index[0]
    receivers = edge_index[1]

    wi0 = params["interaction_encoder"]["W"][0]
    wa, wb, wc = wi0[:LAT], wi0[LAT:2 * LAT], wi0[2 * LAT:]

    decs = [_dec_weights_n(params[n]) for n in
            ("node_weight_decoder", "m_decoder", "i_decoder",
             "f_ext_decoder", "t_ext_decoder")]
    aux = _node_stage(node_latent, node_a_t1, node_alpha_t1, decs)
    w = aux[:, 0]

    ge, ws, wr = _sc_gather(node_latent, w, senders, receivers)

    sv = jnp.concatenate([
        vector_a.T, vector_b.T, vector_c.T,
        senders_v_t_.T, senders_w_t_.T,
        receivers_v_t_.T, receivers_w_t_.T,
        edge_dx_.T, edge_attr.T,
        senders_pos.T, receivers_pos.T,
        ws[None], wr[None],
        jnp.zeros((4, NE), F32)], axis=0)

    pi = params["interaction_encoder"]
    wlist = (_enc_weights(params["edge_feat_encoder"])
             + _enc_weights(params["node_feat_encoder"])
             + [wa, wb, wc, _row_bias(pi["b"][0]), pi["W"][1],
                _row_bias(pi["b"][1]), pi["W"][2], _row_bias(pi["b"][2]),
                _row_bias(pi["ln_g"]), _row_bias(pi["ln_b"])]
             + _dec_weights(params["i1_decoder"])
             + _dec_weights(params["i2_decoder"])
             + _dec_weights(params["f_scaler"]))
    il, o8 = _edge_stage(sv, ge, wlist)

    ft = o8.T
    offs = (receivers[:, None] * 8
            + jnp.arange(8, dtype=jnp.int32)[None]).reshape(-1)
    part = _sc_scatter(ft.reshape(-1), offs)
    part = part.reshape(2, NNP, 8)[:, :NN, :]
    rl, ra = _finalize(part, aux)
    return (rl, ra, il)


# speedup vs baseline: 2.1963x; 1.0571x over previous
import functools

import jax
import jax.numpy as jnp
from jax import lax
from jax.experimental import pallas as pl
from jax.experimental.pallas import tpu as pltpu
from jax.experimental.pallas import tpu_sc as plsc

F32 = jnp.float32
LAT = 128
NN = 10000
NE = 320000
EB = 1280
NB = 2000
NWORK = 32
EPW = NE // NWORK
CH = 80
NCHUNK = EPW // CH


def _relu(x):
    return jnp.maximum(x, 0.0)


def _ln(x, g, b):
    mu = jnp.mean(x, axis=-1, keepdims=True)
    xc = x - mu
    var = jnp.mean(xc * xc, axis=-1, keepdims=True)
    return xc * (lax.rsqrt(var + 1e-5) * g) + b


def _mm(x, w, prec=None):
    return lax.dot_general(x, w, (((1,), (0,)), ((), ())),
                           preferred_element_type=F32, precision=prec)


def _mmT(xt, w, prec=None):
    return lax.dot_general(xt, w, (((0,), (0,)), ((), ())),
                           preferred_element_type=F32, precision=prec)


def _mmO(w, h, prec=None):
    return lax.dot_general(w, h, (((0,), (1,)), ((), ())),
                           preferred_element_type=F32, precision=prec)


def _node_body(nl_ref, na_ref, nal_ref,
               nw0, nb0, nw1, nb1, nw2, nb2,
               mw0, mb0, mw1, mb1, mw2, mb2,
               iw0, ib0, iw1, ib1, iw2, ib2,
               fw0, fb0, fw1, fb1, fw2, fb2,
               tw0, tb0, tw1, tb1, tw2, tb2,
               aux_ref):
    x = nl_ref[...]

    def dec(w0, b0, w1, b1, w2, b2):
        h = _relu(_mm(x, w0[...]) + b0[...])
        h = _relu(_mm(h, w1[...]) + b1[...])
        return _mm(h, w2[...]) + b2[...]

    w = dec(nw0, nb0, nw1, nb1, nw2, nb2)
    m = dec(mw0, mb0, mw1, mb1, mw2, mb2)
    mi = dec(iw0, ib0, iw1, ib1, iw2, ib2)
    fx = dec(fw0, fb0, fw1, fb1, fw2, fb2)
    tx = dec(tw0, tb0, tw1, tb1, tw2, tb2)
    bl = m * na_ref[...] - fx
    ba = mi * nal_ref[...] - tx
    aux_ref[...] = jnp.concatenate([w, bl, ba, jnp.zeros_like(w)], axis=1)


def _node_stage(node_latent, node_a_t1, node_alpha_t1, decs):
    grid = NN // NB
    full = lambda a: pl.BlockSpec(a.shape, lambda i: (0,) * a.ndim)
    win = [
        pl.BlockSpec((NB, LAT), lambda i: (i, 0)),
        pl.BlockSpec((NB, 3), lambda i: (i, 0)),
        pl.BlockSpec((NB, 3), lambda i: (i, 0)),
    ]
    flat = []
    for d in decs:
        flat += d
    win += [full(a) for a in flat]
    return pl.pallas_call(
        _node_body,
        grid=(grid,),
        in_specs=win,
        out_specs=pl.BlockSpec((NB, 8), lambda i: (i, 0)),
        out_shape=jax.ShapeDtypeStruct((NN, 8), F32),
    )(node_latent, node_a_t1, node_alpha_t1, *flat)


def _sc_gather(g, w, snd, rcv):
    mesh = plsc.VectorSubcoreMesh(core_axis_name="c", subcore_axis_name="s")

    @functools.partial(
        pl.kernel, mesh=mesh,
        out_type=[jax.ShapeDtypeStruct((NE, LAT), F32),
                  jax.ShapeDtypeStruct((NE,), F32),
                  jax.ShapeDtypeStruct((NE,), F32)],
        scratch_types=[pltpu.VMEM((CH,), jnp.int32),
                       pltpu.VMEM((CH,), jnp.int32),
                       pltpu.VMEM((CH, LAT), F32),
                       pltpu.VMEM((CH, LAT), F32),
                       pltpu.VMEM((NN,), F32),
                       pltpu.VMEM((CH,), F32),
                       pltpu.VMEM((CH,), F32),
                       pltpu.SemaphoreType.DMA],
        compiler_params=pltpu.CompilerParams(needs_layout_passes=False),
    )
    def k(g_hbm, w_hbm, snd_hbm, rcv_hbm, ge_hbm, ws_hbm, wr_hbm,
          idx_s, idx_r, rows_s, rows_r, wbuf, wsv, wrv, sem):
        wid = lax.axis_index("s") * 2 + lax.axis_index("c")
        base0 = wid * EPW
        pltpu.sync_copy(w_hbm, wbuf)

        def chunk(kk, carry):
            base = base0 + kk * CH
            pltpu.sync_copy(snd_hbm.at[pl.ds(base, CH)], idx_s)
            pltpu.sync_copy(rcv_hbm.at[pl.ds(base, CH)], idx_r)
            cp1 = pltpu.async_copy(g_hbm.at[idx_s], rows_s, sem)
            cp2 = pltpu.async_copy(g_hbm.at[idx_r], rows_r, sem)
            cp1.wait()
            cp2.wait()

            def addrow(i, c2):
                for j in range(LAT // 16):
                    sl = pl.ds(j * 16, 16)
                    rows_s[i, sl] = rows_s[i, sl] + rows_r[i, sl]
                return c2

            lax.fori_loop(0, CH, addrow, 0)
            for k2 in range(CH // 16):
                sl = pl.ds(k2 * 16, 16)
                wsv[sl] = plsc.load_gather(wbuf, [idx_s[sl]])
                wrv[sl] = plsc.load_gather(wbuf, [idx_r[sl]])
            pltpu.sync_copy(rows_s, ge_hbm.at[pl.ds(base, CH)])
            pltpu.sync_copy(wsv, ws_hbm.at[pl.ds(base, CH)])
            pltpu.sync_copy(wrv, wr_hbm.at[pl.ds(base, CH)])
            return carry

        lax.fori_loop(0, NCHUNK, chunk, 0)

    return k(g, w, snd, rcv)


NNP = 10240
TBL = NNP * 8
NQ = 8
QL = TBL // NQ
SLC = QL // 16
CE = 128
NCK = NE // CE


def _sc_scatter(o8, rcv):
    mesh = plsc.VectorSubcoreMesh(core_axis_name="c", subcore_axis_name="s")

    @functools.partial(
        pl.kernel, mesh=mesh,
        out_type=jax.ShapeDtypeStruct((2, TBL), F32),
        scratch_types=[pltpu.VMEM((8, CE), F32),
                       pltpu.VMEM((CE,), jnp.int32),
                       pltpu.VMEM((TBL,), F32),
                       pltpu.VMEM((SLC,), F32),
                       pltpu.VMEM((SLC,), F32),
                       pltpu.VMEM_SHARED((16, QL), F32),
                       pltpu.SemaphoreType.DMA],
        compiler_params=pltpu.CompilerParams(needs_layout_passes=False),
    )
    def k(o8_hbm, rcv_hbm, part_hbm, fbuf, rbuf, tbl, acc, tmp, shared, sem):
        c = lax.axis_index("c")
        s = lax.axis_index("s")
        wid = s * 2 + c
        z16 = jnp.zeros((16,), F32)
        iot = lax.iota(jnp.int32, 16)
        eidx = lax.shift_right_logical(iot, 3)
        cidx = lax.bitwise_and(iot, 7)

        def zero(i, carry):
            tbl[pl.ds(i * 16, 16)] = z16
            return carry

        lax.fori_loop(0, TBL // 16, zero, 0)
        nck = jnp.where(wid < NCK % NWORK, NCK // NWORK + 1, NCK // NWORK)

        def chunk(kk, carry):
            base = (wid + kk * NWORK) * CE
            pltpu.sync_copy(rcv_hbm.at[pl.ds(base, CE)], rbuf)
            pltpu.sync_copy(o8_hbm.at[:, pl.ds(base, CE)], fbuf)

            def scat16(g, c2):
                ex = eidx + 2 * g
                offs = plsc.load_gather(rbuf, [ex]) * 8 + cidx
                vals = plsc.load_gather(fbuf, [cidx, ex])
                plsc.addupdate_scatter(tbl, [offs], vals)
                return c2

            lax.fori_loop(0, CE // 2, scat16, 0)
            return carry

        lax.fori_loop(0, nck, chunk, 0)

        def red(i, carry):
            sl = pl.ds(i * 16, 16)
            acc[sl] = acc[sl] + tmp[sl]
            return carry

        for q in range(NQ):
            qoff = q * QL
            pltpu.sync_copy(tbl.at[pl.ds(qoff, QL)], shared.at[s])
            plsc.subcore_barrier()
            myslc = pl.ds(s * SLC, SLC)
            pltpu.sync_copy(shared.at[0, myslc], acc)
            for t in range(1, 16):
                pltpu.sync_copy(shared.at[t, myslc], tmp)
                lax.fori_loop(0, SLC // 16, red, 0)
            pltpu.sync_copy(acc, part_hbm.at[c, pl.ds(qoff + s * SLC, SLC)])
            plsc.subcore_barrier()

    return k(o8, rcv)


def _edge_body(svn, ge,
               ew0, eb0, ew1, eb1, ew2, eb2, elg, elb,
               nw0, nb0, nw1, nb1, nw2, nb2, nlg, nlb,
               wa, wb, wc, ib0, iw1, ib1, iw2, ib2, ilg, ilb,
               aw0, ab0, aw1, ab1, aw2, ab2,
               cw0, cb0, cw1, cb1, cw2, cb2,
               sw0, sb0, sw1, sb1, sw2, sb2,
               il_ref, o8_ref):
    sv = jnp.transpose(svn[...], (1, 0))
    r = lambda i: sv[i:i + 1, :]

    def dot3(a0, a1, a2, b0, b1, b2):
        return a0 * b0 + a1 * b1 + a2 * b2

    def project(v0, v1, v2, sign):
        p0 = dot3(r(0), r(1), r(2), v0, v1, v2) * sign
        p1 = dot3(r(3), r(4), r(5), v0, v1, v2) * sign
        p2 = dot3(r(6), r(7), r(8), v0, v1, v2) * sign
        return p0, p1, p2

    sv0, sv1, sv2 = project(r(9), r(10), r(11), 1.0)
    sw_0, sw_1, sw_2 = project(r(12), r(13), r(14), 1.0)
    rv0, rv1, rv2 = project(r(15), r(16), r(17), -1.0)
    rw_0, rw_1, rw_2 = project(r(18), r(19), r(20), -1.0)

    z = jnp.zeros_like(sv0)
    sfT = jnp.concatenate([sv0, sv1, sv2, sw_0, sw_1, sw_2, z, z], axis=0)
    rfT = jnp.concatenate([rv0, rv1, rv2, rw_0, rw_1, rw_2, z, z], axis=0)
    nrm = jnp.sqrt(r(21) * r(21) + r(22) * r(22) + r(23) * r(23))
    efT = jnp.concatenate([nrm, sv[24:28, :], z, z, z], axis=0)

    def enc(xT, w0, b0, w1, b1, w2, b2, lg, lb):
        h = _relu(_mmT(xT, w0[...]) + b0[...])
        h = _relu(_mm(h, w1[...]) + b1[...])
        h = _mm(h, w2[...]) + b2[...]
        return _ln(h, lg[...], lb[...])

    sp = enc(sfT, nw0, nb0, nw1, nb1, nw2, nb2, nlg, nlb)
    rp = enc(rfT, nw0, nb0, nw1, nb1, nw2, nb2, nlg, nlb)
    el = enc(efT, ew0, eb0, ew1, eb1, ew2, eb2, elg, elb)

    x1 = _relu(_mm(sp + rp, wa[...]) + _mm(ge[...], wb[...])
               + _mm(el, wc[...]) + ib0[...])
    x2 = _relu(_mm(x1, iw1[...]) + ib1[...])
    il = _ln(_mm(x2, iw2[...]) + ib2[...], ilg[...], ilb[...])
    il_ref[...] = il

    def dec(w0, b0, w1, b1, w2, b2):
        h = _relu(_mm(il, w0[...]) + b0[...])
        h = _relu(_mm(h, w1[...]) + b1[...])
        return _mmO(w2[...], h) + b2[...]

    cf = dec(aw0, ab0, aw1, ab1, aw2, ab2)
    ca = dec(cw0, cb0, cw1, cb1, cw2, cb2)
    lam = dec(sw0, sb0, sw1, sb1, sw2, sb2)[0:1, :]

    def combo(cT, i):
        c0, c1, c2 = cT[0:1, :], cT[1:2, :], cT[2:3, :]
        return c0 * r(i) + c1 * r(3 + i) + c2 * r(6 + i)

    f0, f1, f2 = combo(cf, 0), combo(cf, 1), combo(cf, 2)
    a0, a1, a2 = combo(ca, 0), combo(ca, 1), combo(ca, 2)

    ws_ = r(34)
    wr_ = r(35)
    den = ws_ + wr_
    lv = []
    for i in range(3):
        r0 = (ws_ * r(28 + i) + wr_ * r(31 + i)) / den
        lv.append(r(31 + i) - r0)
    g0, g1, g2 = f0 * lam, f1 * lam, f2 * lam
    t0 = lv[1] * g2 - lv[2] * g1
    t1 = lv[2] * g0 - lv[0] * g2
    t2 = lv[0] * g1 - lv[1] * g0
    o8_ref[...] = jnp.concatenate(
        [f0, f1, f2, a0 - t0, a1 - t1, a2 - t2, z, z], axis=0)


def _edge_stage(sv, ge, wlist):
    grid = NE // EB
    full = lambda a: pl.BlockSpec(a.shape, lambda i: (0,) * a.ndim)
    in_specs = [pl.BlockSpec((EB, 40), lambda i: (i, 0)),
                pl.BlockSpec((EB, LAT), lambda i: (i, 0))]
    in_specs += [full(a) for a in wlist]
    return pl.pallas_call(
        _edge_body,
        grid=(grid,),
        in_specs=in_specs,
        out_specs=[pl.BlockSpec((EB, LAT), lambda i: (i, 0)),
                   pl.BlockSpec((8, EB), lambda i: (0, i))],
        out_shape=[jax.ShapeDtypeStruct((NE, LAT), F32),
                   jax.ShapeDtypeStruct((8, NE), F32)],
    )(sv, ge, *wlist)


def _fin_body(part_ref, aux_ref, rl_ref, ra_ref):
    p0 = part_ref[0]
    p1 = part_ref[1]
    rl_ref[...] = aux_ref[:, 1:4] - p0[:, 0:3] - p1[:, 0:3]
    ra_ref[...] = aux_ref[:, 4:7] - p0[:, 3:6] - p1[:, 3:6]


def _finalize(part, aux):
    grid = NN // NB
    return pl.pallas_call(
        _fin_body,
        grid=(grid,),
        in_specs=[pl.BlockSpec((2, NB, 8), lambda i: (0, i, 0)),
                  pl.BlockSpec((NB, 8), lambda i: (i, 0))],
        out_specs=[pl.BlockSpec((NB, 3), lambda i: (i, 0)),
                   pl.BlockSpec((NB, 3), lambda i: (i, 0))],
        out_shape=[jax.ShapeDtypeStruct((NN, 3), F32),
                   jax.ShapeDtypeStruct((NN, 3), F32)],
    )(part, aux)


def _row_bias(b):
    return b.reshape(1, -1)


def _enc_weights(p):
    w0 = p["W"][0]
    w0p = jnp.zeros((8, LAT), F32).at[: w0.shape[0]].set(w0)
    return [w0p, _row_bias(p["b"][0]), p["W"][1], _row_bias(p["b"][1]),
            p["W"][2], _row_bias(p["b"][2]),
            _row_bias(p["ln_g"]), _row_bias(p["ln_b"])]


def _dec_weights(p):
    w2 = p["W"][2]
    w2p = jnp.zeros((LAT, 8), F32).at[:, : w2.shape[1]].set(w2)
    b2p = jnp.zeros((8,), F32).at[: w2.shape[1]].set(p["b"][2]).reshape(8, 1)
    return [p["W"][0], _row_bias(p["b"][0]),
            p["W"][1], _row_bias(p["b"][1]), w2p, b2p]


def _dec_weights_n(p):
    return [p["W"][0], _row_bias(p["b"][0]), p["W"][1], _row_bias(p["b"][1]),
            p["W"][2], _row_bias(p["b"][2])]


def kernel(edge_index, senders_pos, receivers_pos, edge_dx_, edge_attr,
           vector_a, vector_b, vector_c, senders_v_t_, senders_v_tm1_,
           senders_w_t_, receivers_v_t_, receivers_v_tm1_, receivers_w_t_,
           node_a_t1, node_alpha_t1, node_latent, params):
    senders = edge_index[0]
    receivers = edge_index[1]

    wi0 = params["interaction_encoder"]["W"][0]
    wa, wb, wc = wi0[:LAT], wi0[LAT:2 * LAT], wi0[2 * LAT:]

    decs = [_dec_weights_n(params[n]) for n in
            ("node_weight_decoder", "m_decoder", "i_decoder",
             "f_ext_decoder", "t_ext_decoder")]
    aux = _node_stage(node_latent, node_a_t1, node_alpha_t1, decs)
    w = aux[:, 0]

    ge, ws, wr = _sc_gather(node_latent, w, senders, receivers)

    sv = jnp.concatenate([
        vector_a, vector_b, vector_c,
        senders_v_t_, senders_w_t_,
        receivers_v_t_, receivers_w_t_,
        edge_dx_, edge_attr,
        senders_pos, receivers_pos,
        ws[:, None], wr[:, None],
        jnp.zeros((NE, 4), F32)], axis=1)

    pi = params["interaction_encoder"]
    wlist = (_enc_weights(params["edge_feat_encoder"])
             + _enc_weights(params["node_feat_encoder"])
             + [wa, wb, wc, _row_bias(pi["b"][0]), pi["W"][1],
                _row_bias(pi["b"][1]), pi["W"][2], _row_bias(pi["b"][2]),
                _row_bias(pi["ln_g"]), _row_bias(pi["ln_b"])]
             + _dec_weights(params["i1_decoder"])
             + _dec_weights(params["i2_decoder"])
             + _dec_weights(params["f_scaler"]))
    il, o8 = _edge_stage(sv, ge, wlist)

    part = _sc_scatter(o8, receivers)
    part = part.reshape(2, NNP, 8)[:, :NN, :]
    rl, ra = _finalize(part, aux)
    return (rl, ra, il)
